# baseline (speedup 1.0000x reference)
import jax
import jax.numpy as jnp
from jax import lax
from jax.experimental import pallas as pl
from jax.experimental.pallas import tpu as pltpu
from jax.experimental.pallas import tpu_sc as plsc

B, NG, NH, FH = 4, 4096, 4096, 8192
EPS = 1e-07
MAX_DIST = 5.0
QS, QL = 32, 128
UNROLL = 4

_NC, _NS, _L = 2, 16, 16
_NW = _NC * _NS
_BF = B * FH
_BV = B * NH
_BQ = B * NG
_FPW = _BF // _NW
_QPW = _BQ // _NW
_HF = _BF // _NC
_HV = _BV // _NC
_VPS = _HV // _NS

_mesh = plsc.VectorSubcoreMesh(core_axis_name="c", subcore_axis_name="s")


def _a1_body(hx, hy, hz, f0, f1, f2, out, f0_v, f1_v, f2_v, g_v):
    cid = lax.axis_index("c")
    sid = lax.axis_index("s")
    wid = sid * _NC + cid
    off = wid * _FPW
    pltpu.sync_copy(f0.at[pl.ds(off, _FPW)], f0_v)
    pltpu.sync_copy(f1.at[pl.ds(off, _FPW)], f1_v)
    pltpu.sync_copy(f2.at[pl.ds(off, _FPW)], f2_v)
    for vi, fv in ((0, f0_v), (1, f1_v), (2, f2_v)):
        for ci, tab in ((0, hx), (1, hy), (2, hz)):
            pltpu.sync_copy(tab.at[fv], g_v)
            pltpu.sync_copy(g_v, out.at[pl.ds((vi * 3 + ci) * _BF + off, _FPW)])


def _a1_call(hx, hy, hz, f0, f1, f2):
    k = pl.kernel(
        _a1_body,
        out_type=jax.ShapeDtypeStruct((9 * _BF,), jnp.float32),
        mesh=_mesh,
        scratch_types=[
            pltpu.VMEM((_FPW,), jnp.int32),
            pltpu.VMEM((_FPW,), jnp.int32),
            pltpu.VMEM((_FPW,), jnp.int32),
            pltpu.VMEM((_FPW,), jnp.float32),
        ],
    )
    return k(hx, hy, hz, f0, f1, f2)


def _a2_body(v_ref, fn_ref):
    v0x, v0y, v0z = v_ref[0], v_ref[1], v_ref[2]
    v1x, v1y, v1z = v_ref[3], v_ref[4], v_ref[5]
    v2x, v2y, v2z = v_ref[6], v_ref[7], v_ref[8]
    e1x, e1y, e1z = v1x - v0x, v1y - v0y, v1z - v0z
    e2x, e2y, e2z = v2x - v0x, v2y - v0y, v2z - v0z
    nx = e1y * e2z - e1z * e2y
    ny = e1z * e2x - e1x * e2z
    nz = e1x * e2y - e1y * e2x
    r = jnp.sqrt(nx * nx + ny * ny + nz * nz) + EPS
    fn_ref[0] = nx / r
    fn_ref[1] = ny / r
    fn_ref[2] = nz / r


def _a2_call(v):
    return pl.pallas_call(
        _a2_body,
        out_shape=jax.ShapeDtypeStruct((3, _BF), jnp.float32),
    )(v.reshape(9, _BF))


def _a3_body(fn, f0, f1, f2, z, out,
             f0_v, f1_v, f2_v, vx_v, vy_v, vz_v, o_v, nx_sp, ny_sp, nz_sp):
    cid = lax.axis_index("c")
    sid = lax.axis_index("s")
    voff = cid * _HV + sid * _VPS
    vsl = pl.ds(voff, _VPS)
    pltpu.sync_copy(z.at[vsl], nx_sp.at[vsl])
    pltpu.sync_copy(z.at[vsl], ny_sp.at[vsl])
    pltpu.sync_copy(z.at[vsl], nz_sp.at[vsl])
    plsc.subcore_barrier()
    fpw = _HF // _NS
    foff = cid * _HF + sid * fpw
    fsl = pl.ds(foff, fpw)
    pltpu.sync_copy(f0.at[fsl], f0_v)
    pltpu.sync_copy(f1.at[fsl], f1_v)
    pltpu.sync_copy(f2.at[fsl], f2_v)
    pltpu.sync_copy(fn.at[pl.ds(0 * _BF + foff, fpw)], vx_v)
    pltpu.sync_copy(fn.at[pl.ds(1 * _BF + foff, fpw)], vy_v)
    pltpu.sync_copy(fn.at[pl.ds(2 * _BF + foff, fpw)], vz_v)
    for fv in (f0_v, f1_v, f2_v):
        pltpu.sync_copy(vx_v, nx_sp.at[fv], add=True)
        pltpu.sync_copy(vy_v, ny_sp.at[fv], add=True)
        pltpu.sync_copy(vz_v, nz_sp.at[fv], add=True)
    plsc.subcore_barrier()
    for ci, sp in ((0, nx_sp), (1, ny_sp), (2, nz_sp)):
        pltpu.sync_copy(sp.at[vsl], o_v)
        pltpu.sync_copy(o_v, out.at[pl.ds(ci * _BV + voff, _VPS)])


def _a3_call(fn, f0, f1, f2):
    k = pl.kernel(
        _a3_body,
        out_type=jax.ShapeDtypeStruct((3 * _BV,), jnp.float32),
        mesh=_mesh,
        scratch_types=[
            pltpu.VMEM((_HF // _NS,), jnp.int32),
            pltpu.VMEM((_HF // _NS,), jnp.int32),
            pltpu.VMEM((_HF // _NS,), jnp.int32),
            pltpu.VMEM((_HF // _NS,), jnp.float32),
            pltpu.VMEM((_HF // _NS,), jnp.float32),
            pltpu.VMEM((_HF // _NS,), jnp.float32),
            pltpu.VMEM((_VPS,), jnp.float32),
            pltpu.VMEM_SHARED((_BV,), jnp.float32),
            pltpu.VMEM_SHARED((_BV,), jnp.float32),
            pltpu.VMEM_SHARED((_BV,), jnp.float32),
        ],
    )
    return k(fn.reshape(-1), f0, f1, f2, jnp.zeros((_BV,), jnp.float32))


def _knn_body(q_ref, a_ref, idx_ref):
    b = pl.program_id(0)
    qx = q_ref[0, 0]
    qy = q_ref[0, 1]
    qz = q_ref[0, 2]

    def step(j, carry):
        best, bidx = carry
        for k in range(UNROLL):
            jj = j * UNROLL + k
            ax = a_ref[0, 0, jj]
            ay = a_ref[0, 1, jj]
            az = a_ref[0, 2, jj]
            c = a_ref[0, 3, jj]
            s = c - 2.0 * (qx * ax + qy * ay + qz * az)
            pred = s < best
            best = jnp.where(pred, s, best)
            bidx = jnp.where(pred, jj, bidx)
        return best, bidx

    best0 = jnp.full((QS, QL), jnp.inf, dtype=jnp.float32)
    bidx0 = jnp.zeros((QS, QL), dtype=jnp.int32)
    _, bidx = lax.fori_loop(0, NH // UNROLL, step, (best0, bidx0))
    idx_ref[0] = bidx + b * NH


def _knn_call(q, a):
    return pl.pallas_call(
        _knn_body,
        grid=(B,),
        in_specs=[
            pl.BlockSpec((1, 3, QS, QL), lambda b: (b, 0, 0, 0)),
            pl.BlockSpec((1, 4, NH), lambda b: (b, 0, 0),
                         memory_space=pltpu.SMEM),
        ],
        out_specs=pl.BlockSpec((1, QS, QL), lambda b: (b, 0, 0)),
        out_shape=jax.ShapeDtypeStruct((B, QS, QL), jnp.int32),
    )(q, a)


def _c1_body(hx, hy, hz, nx, ny, nz, idx, out, i_v, g_v):
    cid = lax.axis_index("c")
    sid = lax.axis_index("s")
    wid = sid * _NC + cid
    off = wid * _QPW
    pltpu.sync_copy(idx.at[pl.ds(off, _QPW)], i_v)
    for ci, tab in ((0, hx), (1, hy), (2, hz), (3, nx), (4, ny), (5, nz)):
        pltpu.sync_copy(tab.at[i_v], g_v)
        pltpu.sync_copy(g_v, out.at[pl.ds(ci * _BQ + off, _QPW)])


def _c1_call(hx, hy, hz, vn, idx):
    k = pl.kernel(
        _c1_body,
        out_type=jax.ShapeDtypeStruct((6 * _BQ,), jnp.float32),
        mesh=_mesh,
        scratch_types=[
            pltpu.VMEM((_QPW,), jnp.int32),
            pltpu.VMEM((_QPW,), jnp.float32),
        ],
    )
    return k(hx, hy, hz, vn[:_BV], vn[_BV:2 * _BV], vn[2 * _BV:], idx)


def _c2_body(g_ref, q_ref, o_ref):
    ax, ay, az = g_ref[0, 0], g_ref[0, 1], g_ref[0, 2]
    nx, ny, nz = g_ref[0, 3], g_ref[0, 4], g_ref[0, 5]
    dx = q_ref[0, 0] - ax
    dy = q_ref[0, 1] - ay
    dz = q_ref[0, 2] - az
    l2 = jnp.sqrt(dx * dx + dy * dy + dz * dz)
    dot = dx * nx + dy * ny + dz * nz
    coll = (dot < 0.0) & (l2 <= MAX_DIST)
    o_ref[pl.program_id(0), 0] = jnp.sum(coll.astype(jnp.float32)) / NG


def _c2_call(g, q_soa):
    return pl.pallas_call(
        _c2_body,
        grid=(B,),
        in_specs=[
            pl.BlockSpec((1, 6, NG), lambda b: (b, 0, 0)),
            pl.BlockSpec((1, 3, NG), lambda b: (b, 0, 0)),
        ],
        out_specs=pl.BlockSpec((B, 1), lambda b: (0, 0),
                               memory_space=pltpu.SMEM),
        out_shape=jax.ShapeDtypeStruct((B, 1), jnp.float32),
    )(g, q_soa)


def kernel(pred, h_state, faces, h_faces):
    del faces
    hx = h_state[:, :, 0].reshape(-1)
    hy = h_state[:, :, 1].reshape(-1)
    hz = h_state[:, :, 2].reshape(-1)
    fadj = h_faces + (jnp.arange(B, dtype=jnp.int32) * NH)[:, None, None]
    f0 = fadj[:, :, 0].reshape(-1)
    f1 = fadj[:, :, 1].reshape(-1)
    f2 = fadj[:, :, 2].reshape(-1)

    v9 = _a1_call(hx, hy, hz, f0, f1, f2)
    fn = _a2_call(v9)
    vn = _a3_call(fn, f0, f1, f2)

    qr = pred.astype(jnp.bfloat16).astype(jnp.float32)
    ar = h_state.astype(jnp.bfloat16).astype(jnp.float32)
    q = qr.transpose(0, 2, 1).reshape(B, 3, QS, QL)
    an2 = jnp.sum(h_state * h_state, axis=-1)
    a = jnp.concatenate([ar.transpose(0, 2, 1), an2[:, None, :]], axis=1)
    idx = _knn_call(q, a).reshape(-1)

    g = _c1_call(hx, hy, hz, vn, idx)
    q_soa = pred.transpose(0, 2, 1)
    return _c2_call(g.reshape(6, B, NG).transpose(1, 0, 2), q_soa)


# speedup vs baseline: 1.1190x; 1.1190x over previous
import jax
import jax.numpy as jnp
from jax import lax
from jax.experimental import pallas as pl
from jax.experimental.pallas import tpu as pltpu
from jax.experimental.pallas import tpu_sc as plsc

B, NG, NH, FH = 4, 4096, 4096, 8192
EPS = 1e-07
MAX_DIST = 5.0
QS, QL = 32, 128
UNROLL = 8

_NC, _NS, _L = 2, 16, 16
_NW = _NC * _NS
_BF = B * FH
_BV = B * NH
_BQ = B * NG
_FPW = _BF // _NW
_QPW = _BQ // _NW
_HF = _BF // _NC
_HV = _BV // _NC
_VPS = _HV // _NS

_mesh = plsc.VectorSubcoreMesh(core_axis_name="c", subcore_axis_name="s")


def _a1_body(hx, hy, hz, f0, f1, f2, out, f0_v, f1_v, f2_v, g_v):
    cid = lax.axis_index("c")
    sid = lax.axis_index("s")
    wid = sid * _NC + cid
    off = wid * _FPW
    pltpu.sync_copy(f0.at[pl.ds(off, _FPW)], f0_v)
    pltpu.sync_copy(f1.at[pl.ds(off, _FPW)], f1_v)
    pltpu.sync_copy(f2.at[pl.ds(off, _FPW)], f2_v)
    for vi, fv in ((0, f0_v), (1, f1_v), (2, f2_v)):
        for ci, tab in ((0, hx), (1, hy), (2, hz)):
            pltpu.sync_copy(tab.at[fv], g_v)
            pltpu.sync_copy(g_v, out.at[pl.ds((vi * 3 + ci) * _BF + off, _FPW)])


def _a1_call(hx, hy, hz, f0, f1, f2):
    k = pl.kernel(
        _a1_body,
        out_type=jax.ShapeDtypeStruct((9 * _BF,), jnp.float32),
        mesh=_mesh,
        scratch_types=[
            pltpu.VMEM((_FPW,), jnp.int32),
            pltpu.VMEM((_FPW,), jnp.int32),
            pltpu.VMEM((_FPW,), jnp.int32),
            pltpu.VMEM((_FPW,), jnp.float32),
        ],
    )
    return k(hx, hy, hz, f0, f1, f2)


def _a2_body(v_ref, fn_ref):
    v0x, v0y, v0z = v_ref[0], v_ref[1], v_ref[2]
    v1x, v1y, v1z = v_ref[3], v_ref[4], v_ref[5]
    v2x, v2y, v2z = v_ref[6], v_ref[7], v_ref[8]
    e1x, e1y, e1z = v1x - v0x, v1y - v0y, v1z - v0z
    e2x, e2y, e2z = v2x - v0x, v2y - v0y, v2z - v0z
    nx = e1y * e2z - e1z * e2y
    ny = e1z * e2x - e1x * e2z
    nz = e1x * e2y - e1y * e2x
    r = jnp.sqrt(nx * nx + ny * ny + nz * nz) + EPS
    fn_ref[0] = nx / r
    fn_ref[1] = ny / r
    fn_ref[2] = nz / r


def _a2_call(v):
    return pl.pallas_call(
        _a2_body,
        out_shape=jax.ShapeDtypeStruct((3, _BF), jnp.float32),
    )(v.reshape(9, _BF))


def _a3_body(fn, f0, f1, f2, z, out,
             f0_v, f1_v, f2_v, vx_v, vy_v, vz_v, o_v, nx_sp, ny_sp, nz_sp):
    cid = lax.axis_index("c")
    sid = lax.axis_index("s")
    voff = cid * _HV + sid * _VPS
    vsl = pl.ds(voff, _VPS)
    pltpu.sync_copy(z.at[vsl], nx_sp.at[vsl])
    pltpu.sync_copy(z.at[vsl], ny_sp.at[vsl])
    pltpu.sync_copy(z.at[vsl], nz_sp.at[vsl])
    plsc.subcore_barrier()
    fpw = _HF // _NS
    foff = cid * _HF + sid * fpw
    fsl = pl.ds(foff, fpw)
    pltpu.sync_copy(f0.at[fsl], f0_v)
    pltpu.sync_copy(f1.at[fsl], f1_v)
    pltpu.sync_copy(f2.at[fsl], f2_v)
    pltpu.sync_copy(fn.at[pl.ds(0 * _BF + foff, fpw)], vx_v)
    pltpu.sync_copy(fn.at[pl.ds(1 * _BF + foff, fpw)], vy_v)
    pltpu.sync_copy(fn.at[pl.ds(2 * _BF + foff, fpw)], vz_v)
    for fv in (f0_v, f1_v, f2_v):
        pltpu.sync_copy(vx_v, nx_sp.at[fv], add=True)
        pltpu.sync_copy(vy_v, ny_sp.at[fv], add=True)
        pltpu.sync_copy(vz_v, nz_sp.at[fv], add=True)
    plsc.subcore_barrier()
    for ci, sp in ((0, nx_sp), (1, ny_sp), (2, nz_sp)):
        pltpu.sync_copy(sp.at[vsl], o_v)
        pltpu.sync_copy(o_v, out.at[pl.ds(ci * _BV + voff, _VPS)])


def _a3_call(fn, f0, f1, f2):
    k = pl.kernel(
        _a3_body,
        out_type=jax.ShapeDtypeStruct((3 * _BV,), jnp.float32),
        mesh=_mesh,
        scratch_types=[
            pltpu.VMEM((_HF // _NS,), jnp.int32),
            pltpu.VMEM((_HF // _NS,), jnp.int32),
            pltpu.VMEM((_HF // _NS,), jnp.int32),
            pltpu.VMEM((_HF // _NS,), jnp.float32),
            pltpu.VMEM((_HF // _NS,), jnp.float32),
            pltpu.VMEM((_HF // _NS,), jnp.float32),
            pltpu.VMEM((_VPS,), jnp.float32),
            pltpu.VMEM_SHARED((_BV,), jnp.float32),
            pltpu.VMEM_SHARED((_BV,), jnp.float32),
            pltpu.VMEM_SHARED((_BV,), jnp.float32),
        ],
    )
    return k(fn.reshape(-1), f0, f1, f2, jnp.zeros((_BV,), jnp.float32))


def _knn_body(q_ref, a_ref, idx_ref):
    b = pl.program_id(0)
    qx = q_ref[0, 0]
    qy = q_ref[0, 1]
    qz = q_ref[0, 2]

    def score(jj):
        ax = a_ref[0, 0, jj]
        ay = a_ref[0, 1, jj]
        az = a_ref[0, 2, jj]
        c = a_ref[0, 3, jj]
        return (qx * ax + qy * ay) + (qz * az + c)

    def step(j, carry):
        b0, i0, b1, i1 = carry
        for k in range(0, UNROLL, 2):
            j0 = j * UNROLL + k
            j1 = j0 + 1
            s0 = score(j0)
            s1 = score(j1)
            p0 = s0 < b0
            p1 = s1 < b1
            b0 = jnp.where(p0, s0, b0)
            i0 = jnp.where(p0, j0, i0)
            b1 = jnp.where(p1, s1, b1)
            i1 = jnp.where(p1, j1, i1)
        return b0, i0, b1, i1

    inf = jnp.full((QS, QL), jnp.inf, dtype=jnp.float32)
    zero = jnp.zeros((QS, QL), dtype=jnp.int32)
    b0, i0, b1, i1 = lax.fori_loop(0, NH // UNROLL, step,
                                   (inf, zero, inf, zero))
    pm = b1 < b0
    bidx = jnp.where(pm, i1, i0)
    idx_ref[0] = bidx + b * NH


def _knn_call(q, a):
    return pl.pallas_call(
        _knn_body,
        grid=(B,),
        in_specs=[
            pl.BlockSpec((1, 3, QS, QL), lambda b: (b, 0, 0, 0)),
            pl.BlockSpec((1, 4, NH), lambda b: (b, 0, 0),
                         memory_space=pltpu.SMEM),
        ],
        out_specs=pl.BlockSpec((1, QS, QL), lambda b: (b, 0, 0)),
        out_shape=jax.ShapeDtypeStruct((B, QS, QL), jnp.int32),
    )(q, a)


def _c1_body(hx, hy, hz, nx, ny, nz, idx, out, i_v, g_v):
    cid = lax.axis_index("c")
    sid = lax.axis_index("s")
    wid = sid * _NC + cid
    off = wid * _QPW
    pltpu.sync_copy(idx.at[pl.ds(off, _QPW)], i_v)
    for ci, tab in ((0, hx), (1, hy), (2, hz), (3, nx), (4, ny), (5, nz)):
        pltpu.sync_copy(tab.at[i_v], g_v)
        pltpu.sync_copy(g_v, out.at[pl.ds(ci * _BQ + off, _QPW)])


def _c1_call(hx, hy, hz, vn, idx):
    k = pl.kernel(
        _c1_body,
        out_type=jax.ShapeDtypeStruct((6 * _BQ,), jnp.float32),
        mesh=_mesh,
        scratch_types=[
            pltpu.VMEM((_QPW,), jnp.int32),
            pltpu.VMEM((_QPW,), jnp.float32),
        ],
    )
    return k(hx, hy, hz, vn[:_BV], vn[_BV:2 * _BV], vn[2 * _BV:], idx)


def _c2_body(g_ref, q_ref, o_ref):
    ax, ay, az = g_ref[0, 0], g_ref[0, 1], g_ref[0, 2]
    nx, ny, nz = g_ref[0, 3], g_ref[0, 4], g_ref[0, 5]
    dx = q_ref[0, 0] - ax
    dy = q_ref[0, 1] - ay
    dz = q_ref[0, 2] - az
    l2 = jnp.sqrt(dx * dx + dy * dy + dz * dz)
    dot = dx * nx + dy * ny + dz * nz
    coll = (dot < 0.0) & (l2 <= MAX_DIST)
    o_ref[pl.program_id(0), 0] = jnp.sum(coll.astype(jnp.float32)) / NG


def _c2_call(g, q_soa):
    return pl.pallas_call(
        _c2_body,
        grid=(B,),
        in_specs=[
            pl.BlockSpec((1, 6, NG), lambda b: (b, 0, 0)),
            pl.BlockSpec((1, 3, NG), lambda b: (b, 0, 0)),
        ],
        out_specs=pl.BlockSpec((B, 1), lambda b: (0, 0),
                               memory_space=pltpu.SMEM),
        out_shape=jax.ShapeDtypeStruct((B, 1), jnp.float32),
    )(g, q_soa)


def kernel(pred, h_state, faces, h_faces):
    del faces
    hx = h_state[:, :, 0].reshape(-1)
    hy = h_state[:, :, 1].reshape(-1)
    hz = h_state[:, :, 2].reshape(-1)
    fadj = h_faces + (jnp.arange(B, dtype=jnp.int32) * NH)[:, None, None]
    f0 = fadj[:, :, 0].reshape(-1)
    f1 = fadj[:, :, 1].reshape(-1)
    f2 = fadj[:, :, 2].reshape(-1)

    v9 = _a1_call(hx, hy, hz, f0, f1, f2)
    fn = _a2_call(v9)
    vn = _a3_call(fn, f0, f1, f2)

    qr = pred.astype(jnp.bfloat16).astype(jnp.float32)
    ar = h_state.astype(jnp.bfloat16).astype(jnp.float32)
    q = qr.transpose(0, 2, 1).reshape(B, 3, QS, QL)
    an2 = jnp.sum(h_state * h_state, axis=-1)
    a = jnp.concatenate([(-2.0 * ar).transpose(0, 2, 1), an2[:, None, :]],
                        axis=1)
    idx = _knn_call(q, a).reshape(-1)

    g = _c1_call(hx, hy, hz, vn, idx)
    q_soa = pred.transpose(0, 2, 1)
    return _c2_call(g.reshape(6, B, NG).transpose(1, 0, 2), q_soa)


# speedup vs baseline: 1.1770x; 1.0518x over previous
import jax
import jax.numpy as jnp
from jax import lax
from jax.experimental import pallas as pl
from jax.experimental.pallas import tpu as pltpu
from jax.experimental.pallas import tpu_sc as plsc

B, NG, NH, FH = 4, 4096, 4096, 8192
EPS = 1e-07
MAX_DIST = 5.0
QS, QL = 32, 128
UNROLL = 16

_NC, _NS, _L = 2, 16, 16
_NW = _NC * _NS
_BF = B * FH
_BV = B * NH
_BQ = B * NG
_FPW = _BF // _NW
_QPW = _BQ // _NW
_HF = _BF // _NC
_HV = _BV // _NC
_VPS = _HV // _NS

_mesh = plsc.VectorSubcoreMesh(core_axis_name="c", subcore_axis_name="s")


def _a1_body(hx, hy, hz, f0, f1, f2, out, f0_v, f1_v, f2_v, g_v):
    cid = lax.axis_index("c")
    sid = lax.axis_index("s")
    wid = sid * _NC + cid
    off = wid * _FPW
    pltpu.sync_copy(f0.at[pl.ds(off, _FPW)], f0_v)
    pltpu.sync_copy(f1.at[pl.ds(off, _FPW)], f1_v)
    pltpu.sync_copy(f2.at[pl.ds(off, _FPW)], f2_v)
    for vi, fv in ((0, f0_v), (1, f1_v), (2, f2_v)):
        for ci, tab in ((0, hx), (1, hy), (2, hz)):
            pltpu.sync_copy(tab.at[fv], g_v)
            pltpu.sync_copy(g_v, out.at[pl.ds((vi * 3 + ci) * _BF + off, _FPW)])


def _a1_call(hx, hy, hz, f0, f1, f2):
    k = pl.kernel(
        _a1_body,
        out_type=jax.ShapeDtypeStruct((9 * _BF,), jnp.float32),
        mesh=_mesh,
        scratch_types=[
            pltpu.VMEM((_FPW,), jnp.int32),
            pltpu.VMEM((_FPW,), jnp.int32),
            pltpu.VMEM((_FPW,), jnp.int32),
            pltpu.VMEM((_FPW,), jnp.float32),
        ],
    )
    return k(hx, hy, hz, f0, f1, f2)


def _a2_body(v_ref, fn_ref):
    v0x, v0y, v0z = v_ref[0], v_ref[1], v_ref[2]
    v1x, v1y, v1z = v_ref[3], v_ref[4], v_ref[5]
    v2x, v2y, v2z = v_ref[6], v_ref[7], v_ref[8]
    e1x, e1y, e1z = v1x - v0x, v1y - v0y, v1z - v0z
    e2x, e2y, e2z = v2x - v0x, v2y - v0y, v2z - v0z
    nx = e1y * e2z - e1z * e2y
    ny = e1z * e2x - e1x * e2z
    nz = e1x * e2y - e1y * e2x
    r = jnp.sqrt(nx * nx + ny * ny + nz * nz) + EPS
    fn_ref[0] = nx / r
    fn_ref[1] = ny / r
    fn_ref[2] = nz / r


def _a2_call(v):
    return pl.pallas_call(
        _a2_body,
        out_shape=jax.ShapeDtypeStruct((3, _BF), jnp.float32),
    )(v.reshape(9, _BF))


def _a3_body(fn, f0, f1, f2, z, out,
             f0_v, f1_v, f2_v, vx_v, vy_v, vz_v, o_v, nx_sp, ny_sp, nz_sp):
    cid = lax.axis_index("c")
    sid = lax.axis_index("s")
    voff = cid * _HV + sid * _VPS
    vsl = pl.ds(voff, _VPS)
    pltpu.sync_copy(z.at[vsl], nx_sp.at[vsl])
    pltpu.sync_copy(z.at[vsl], ny_sp.at[vsl])
    pltpu.sync_copy(z.at[vsl], nz_sp.at[vsl])
    plsc.subcore_barrier()
    fpw = _HF // _NS
    foff = cid * _HF + sid * fpw
    fsl = pl.ds(foff, fpw)
    pltpu.sync_copy(f0.at[fsl], f0_v)
    pltpu.sync_copy(f1.at[fsl], f1_v)
    pltpu.sync_copy(f2.at[fsl], f2_v)
    pltpu.sync_copy(fn.at[pl.ds(0 * _BF + foff, fpw)], vx_v)
    pltpu.sync_copy(fn.at[pl.ds(1 * _BF + foff, fpw)], vy_v)
    pltpu.sync_copy(fn.at[pl.ds(2 * _BF + foff, fpw)], vz_v)
    for fv in (f0_v, f1_v, f2_v):
        pltpu.sync_copy(vx_v, nx_sp.at[fv], add=True)
        pltpu.sync_copy(vy_v, ny_sp.at[fv], add=True)
        pltpu.sync_copy(vz_v, nz_sp.at[fv], add=True)
    plsc.subcore_barrier()
    for ci, sp in ((0, nx_sp), (1, ny_sp), (2, nz_sp)):
        pltpu.sync_copy(sp.at[vsl], o_v)
        pltpu.sync_copy(o_v, out.at[pl.ds(ci * _BV + voff, _VPS)])


def _a3_call(fn, f0, f1, f2):
    k = pl.kernel(
        _a3_body,
        out_type=jax.ShapeDtypeStruct((3 * _BV,), jnp.float32),
        mesh=_mesh,
        scratch_types=[
            pltpu.VMEM((_HF // _NS,), jnp.int32),
            pltpu.VMEM((_HF // _NS,), jnp.int32),
            pltpu.VMEM((_HF // _NS,), jnp.int32),
            pltpu.VMEM((_HF // _NS,), jnp.float32),
            pltpu.VMEM((_HF // _NS,), jnp.float32),
            pltpu.VMEM((_HF // _NS,), jnp.float32),
            pltpu.VMEM((_VPS,), jnp.float32),
            pltpu.VMEM_SHARED((_BV,), jnp.float32),
            pltpu.VMEM_SHARED((_BV,), jnp.float32),
            pltpu.VMEM_SHARED((_BV,), jnp.float32),
        ],
    )
    return k(fn.reshape(-1), f0, f1, f2, jnp.zeros((_BV,), jnp.float32))


def _knn_body(q_ref, a_ref, idx_ref):
    b = pl.program_id(0)
    qx = q_ref[0, 0]
    qy = q_ref[0, 1]
    qz = q_ref[0, 2]

    def score(jj):
        ax = a_ref[0, 0, jj]
        ay = a_ref[0, 1, jj]
        az = a_ref[0, 2, jj]
        c = a_ref[0, 3, jj]
        return (qx * ax + qy * ay) + (qz * az + c)

    def step(j, carry):
        b0, i0, b1, i1 = carry
        for k in range(0, UNROLL, 2):
            j0 = j * UNROLL + k
            j1 = j0 + 1
            s0 = score(j0)
            s1 = score(j1)
            p0 = s0 < b0
            p1 = s1 < b1
            b0 = jnp.where(p0, s0, b0)
            i0 = jnp.where(p0, j0, i0)
            b1 = jnp.where(p1, s1, b1)
            i1 = jnp.where(p1, j1, i1)
        return b0, i0, b1, i1

    inf = jnp.full((QS, QL), jnp.inf, dtype=jnp.float32)
    zero = jnp.zeros((QS, QL), dtype=jnp.int32)
    b0, i0, b1, i1 = lax.fori_loop(0, NH // UNROLL, step,
                                   (inf, zero, inf, zero))
    pm = b1 < b0
    bidx = jnp.where(pm, i1, i0)
    idx_ref[0] = bidx + b * NH


def _knn_call(q, a):
    return pl.pallas_call(
        _knn_body,
        grid=(B,),
        in_specs=[
            pl.BlockSpec((1, 3, QS, QL), lambda b: (b, 0, 0, 0)),
            pl.BlockSpec((1, 4, NH), lambda b: (b, 0, 0),
                         memory_space=pltpu.SMEM),
        ],
        out_specs=pl.BlockSpec((1, QS, QL), lambda b: (b, 0, 0)),
        out_shape=jax.ShapeDtypeStruct((B, QS, QL), jnp.int32),
    )(q, a)


def _c1_body(hx, hy, hz, nx, ny, nz, idx, out, i_v, g_v):
    cid = lax.axis_index("c")
    sid = lax.axis_index("s")
    wid = sid * _NC + cid
    off = wid * _QPW
    pltpu.sync_copy(idx.at[pl.ds(off, _QPW)], i_v)
    for ci, tab in ((0, hx), (1, hy), (2, hz), (3, nx), (4, ny), (5, nz)):
        pltpu.sync_copy(tab.at[i_v], g_v)
        pltpu.sync_copy(g_v, out.at[pl.ds(ci * _BQ + off, _QPW)])


def _c1_call(hx, hy, hz, vn, idx):
    k = pl.kernel(
        _c1_body,
        out_type=jax.ShapeDtypeStruct((6 * _BQ,), jnp.float32),
        mesh=_mesh,
        scratch_types=[
            pltpu.VMEM((_QPW,), jnp.int32),
            pltpu.VMEM((_QPW,), jnp.float32),
        ],
    )
    return k(hx, hy, hz, vn[:_BV], vn[_BV:2 * _BV], vn[2 * _BV:], idx)


def _c2_body(g_ref, q_ref, o_ref):
    ax, ay, az = g_ref[0, 0], g_ref[0, 1], g_ref[0, 2]
    nx, ny, nz = g_ref[0, 3], g_ref[0, 4], g_ref[0, 5]
    dx = q_ref[0, 0] - ax
    dy = q_ref[0, 1] - ay
    dz = q_ref[0, 2] - az
    l2 = jnp.sqrt(dx * dx + dy * dy + dz * dz)
    dot = dx * nx + dy * ny + dz * nz
    coll = (dot < 0.0) & (l2 <= MAX_DIST)
    o_ref[pl.program_id(0), 0] = jnp.sum(coll.astype(jnp.float32)) / NG


def _c2_call(g, q_soa):
    return pl.pallas_call(
        _c2_body,
        grid=(B,),
        in_specs=[
            pl.BlockSpec((1, 6, NG), lambda b: (b, 0, 0)),
            pl.BlockSpec((1, 3, NG), lambda b: (b, 0, 0)),
        ],
        out_specs=pl.BlockSpec((B, 1), lambda b: (0, 0),
                               memory_space=pltpu.SMEM),
        out_shape=jax.ShapeDtypeStruct((B, 1), jnp.float32),
    )(g, q_soa)


def kernel(pred, h_state, faces, h_faces):
    del faces
    hx = h_state[:, :, 0].reshape(-1)
    hy = h_state[:, :, 1].reshape(-1)
    hz = h_state[:, :, 2].reshape(-1)
    fadj = h_faces + (jnp.arange(B, dtype=jnp.int32) * NH)[:, None, None]
    f0 = fadj[:, :, 0].reshape(-1)
    f1 = fadj[:, :, 1].reshape(-1)
    f2 = fadj[:, :, 2].reshape(-1)

    v9 = _a1_call(hx, hy, hz, f0, f1, f2)
    fn = _a2_call(v9)
    vn = _a3_call(fn, f0, f1, f2)

    qr = pred.astype(jnp.bfloat16).astype(jnp.float32)
    ar = h_state.astype(jnp.bfloat16).astype(jnp.float32)
    q = qr.transpose(0, 2, 1).reshape(B, 3, QS, QL)
    an2 = jnp.sum(h_state * h_state, axis=-1)
    a = jnp.concatenate([(-2.0 * ar).transpose(0, 2, 1), an2[:, None, :]],
                        axis=1)
    idx = _knn_call(q, a).reshape(-1)

    g = _c1_call(hx, hy, hz, vn, idx)
    q_soa = pred.transpose(0, 2, 1)
    return _c2_call(g.reshape(6, B, NG).transpose(1, 0, 2), q_soa)


# speedup vs baseline: 1.2065x; 1.0251x over previous
import jax
import jax.numpy as jnp
from jax import lax
from jax.experimental import pallas as pl
from jax.experimental.pallas import tpu as pltpu
from jax.experimental.pallas import tpu_sc as plsc

B, NG, NH, FH = 4, 4096, 4096, 8192
EPS = 1e-07
MAX_DIST = 5.0
QS, QL = 32, 128
UNROLL = 32

_NC, _NS, _L = 2, 16, 16
_NW = _NC * _NS
_BF = B * FH
_BV = B * NH
_BQ = B * NG
_FPW = _BF // _NW
_QPW = _BQ // _NW
_HF = _BF // _NC
_HV = _BV // _NC
_VPS = _HV // _NS

_mesh = plsc.VectorSubcoreMesh(core_axis_name="c", subcore_axis_name="s")


def _a1_body(hx, hy, hz, f0, f1, f2, out, f0_v, f1_v, f2_v, g_v):
    cid = lax.axis_index("c")
    sid = lax.axis_index("s")
    wid = sid * _NC + cid
    off = wid * _FPW
    pltpu.sync_copy(f0.at[pl.ds(off, _FPW)], f0_v)
    pltpu.sync_copy(f1.at[pl.ds(off, _FPW)], f1_v)
    pltpu.sync_copy(f2.at[pl.ds(off, _FPW)], f2_v)
    for vi, fv in ((0, f0_v), (1, f1_v), (2, f2_v)):
        for ci, tab in ((0, hx), (1, hy), (2, hz)):
            pltpu.sync_copy(tab.at[fv], g_v)
            pltpu.sync_copy(g_v, out.at[pl.ds((vi * 3 + ci) * _BF + off, _FPW)])


def _a1_call(hx, hy, hz, f0, f1, f2):
    k = pl.kernel(
        _a1_body,
        out_type=jax.ShapeDtypeStruct((9 * _BF,), jnp.float32),
        mesh=_mesh,
        scratch_types=[
            pltpu.VMEM((_FPW,), jnp.int32),
            pltpu.VMEM((_FPW,), jnp.int32),
            pltpu.VMEM((_FPW,), jnp.int32),
            pltpu.VMEM((_FPW,), jnp.float32),
        ],
    )
    return k(hx, hy, hz, f0, f1, f2)


def _a2_body(v_ref, fn_ref):
    v0x, v0y, v0z = v_ref[0], v_ref[1], v_ref[2]
    v1x, v1y, v1z = v_ref[3], v_ref[4], v_ref[5]
    v2x, v2y, v2z = v_ref[6], v_ref[7], v_ref[8]
    e1x, e1y, e1z = v1x - v0x, v1y - v0y, v1z - v0z
    e2x, e2y, e2z = v2x - v0x, v2y - v0y, v2z - v0z
    nx = e1y * e2z - e1z * e2y
    ny = e1z * e2x - e1x * e2z
    nz = e1x * e2y - e1y * e2x
    r = jnp.sqrt(nx * nx + ny * ny + nz * nz) + EPS
    fn_ref[0] = nx / r
    fn_ref[1] = ny / r
    fn_ref[2] = nz / r


def _a2_call(v):
    return pl.pallas_call(
        _a2_body,
        out_shape=jax.ShapeDtypeStruct((3, _BF), jnp.float32),
    )(v.reshape(9, _BF))


def _a3_body(fn, f0, f1, f2, z, out,
             f0_v, f1_v, f2_v, vx_v, vy_v, vz_v, o_v, nx_sp, ny_sp, nz_sp):
    cid = lax.axis_index("c")
    sid = lax.axis_index("s")
    voff = cid * _HV + sid * _VPS
    vsl = pl.ds(voff, _VPS)
    pltpu.sync_copy(z.at[vsl], nx_sp.at[vsl])
    pltpu.sync_copy(z.at[vsl], ny_sp.at[vsl])
    pltpu.sync_copy(z.at[vsl], nz_sp.at[vsl])
    plsc.subcore_barrier()
    fpw = _HF // _NS
    foff = cid * _HF + sid * fpw
    fsl = pl.ds(foff, fpw)
    pltpu.sync_copy(f0.at[fsl], f0_v)
    pltpu.sync_copy(f1.at[fsl], f1_v)
    pltpu.sync_copy(f2.at[fsl], f2_v)
    pltpu.sync_copy(fn.at[pl.ds(0 * _BF + foff, fpw)], vx_v)
    pltpu.sync_copy(fn.at[pl.ds(1 * _BF + foff, fpw)], vy_v)
    pltpu.sync_copy(fn.at[pl.ds(2 * _BF + foff, fpw)], vz_v)
    for fv in (f0_v, f1_v, f2_v):
        pltpu.sync_copy(vx_v, nx_sp.at[fv], add=True)
        pltpu.sync_copy(vy_v, ny_sp.at[fv], add=True)
        pltpu.sync_copy(vz_v, nz_sp.at[fv], add=True)
    plsc.subcore_barrier()
    for ci, sp in ((0, nx_sp), (1, ny_sp), (2, nz_sp)):
        pltpu.sync_copy(sp.at[vsl], o_v)
        pltpu.sync_copy(o_v, out.at[pl.ds(ci * _BV + voff, _VPS)])


def _a3_call(fn, f0, f1, f2):
    k = pl.kernel(
        _a3_body,
        out_type=jax.ShapeDtypeStruct((3 * _BV,), jnp.float32),
        mesh=_mesh,
        scratch_types=[
            pltpu.VMEM((_HF // _NS,), jnp.int32),
            pltpu.VMEM((_HF // _NS,), jnp.int32),
            pltpu.VMEM((_HF // _NS,), jnp.int32),
            pltpu.VMEM((_HF // _NS,), jnp.float32),
            pltpu.VMEM((_HF // _NS,), jnp.float32),
            pltpu.VMEM((_HF // _NS,), jnp.float32),
            pltpu.VMEM((_VPS,), jnp.float32),
            pltpu.VMEM_SHARED((_BV,), jnp.float32),
            pltpu.VMEM_SHARED((_BV,), jnp.float32),
            pltpu.VMEM_SHARED((_BV,), jnp.float32),
        ],
    )
    return k(fn.reshape(-1), f0, f1, f2, jnp.zeros((_BV,), jnp.float32))


def _knn_body(q_ref, a_ref, idx_ref):
    b = pl.program_id(0)
    qx = q_ref[0, 0]
    qy = q_ref[0, 1]
    qz = q_ref[0, 2]

    def score(jj):
        ax = a_ref[0, 0, jj]
        ay = a_ref[0, 1, jj]
        az = a_ref[0, 2, jj]
        c = a_ref[0, 3, jj]
        return (qx * ax + qy * ay) + (qz * az + c)

    def step(j, carry):
        b0, i0, b1, i1 = carry
        for k in range(0, UNROLL, 2):
            j0 = j * UNROLL + k
            j1 = j0 + 1
            s0 = score(j0)
            s1 = score(j1)
            p0 = s0 < b0
            p1 = s1 < b1
            b0 = jnp.where(p0, s0, b0)
            i0 = jnp.where(p0, j0, i0)
            b1 = jnp.where(p1, s1, b1)
            i1 = jnp.where(p1, j1, i1)
        return b0, i0, b1, i1

    inf = jnp.full((QS, QL), jnp.inf, dtype=jnp.float32)
    zero = jnp.zeros((QS, QL), dtype=jnp.int32)
    b0, i0, b1, i1 = lax.fori_loop(0, NH // UNROLL, step,
                                   (inf, zero, inf, zero))
    pm = b1 < b0
    bidx = jnp.where(pm, i1, i0)
    idx_ref[0] = bidx + b * NH


def _knn_call(q, a):
    return pl.pallas_call(
        _knn_body,
        grid=(B,),
        in_specs=[
            pl.BlockSpec((1, 3, QS, QL), lambda b: (b, 0, 0, 0)),
            pl.BlockSpec((1, 4, NH), lambda b: (b, 0, 0),
                         memory_space=pltpu.SMEM),
        ],
        out_specs=pl.BlockSpec((1, QS, QL), lambda b: (b, 0, 0)),
        out_shape=jax.ShapeDtypeStruct((B, QS, QL), jnp.int32),
    )(q, a)


def _c1_body(hx, hy, hz, nx, ny, nz, idx, out, i_v, g_v):
    cid = lax.axis_index("c")
    sid = lax.axis_index("s")
    wid = sid * _NC + cid
    off = wid * _QPW
    pltpu.sync_copy(idx.at[pl.ds(off, _QPW)], i_v)
    for ci, tab in ((0, hx), (1, hy), (2, hz), (3, nx), (4, ny), (5, nz)):
        pltpu.sync_copy(tab.at[i_v], g_v)
        pltpu.sync_copy(g_v, out.at[pl.ds(ci * _BQ + off, _QPW)])


def _c1_call(hx, hy, hz, vn, idx):
    k = pl.kernel(
        _c1_body,
        out_type=jax.ShapeDtypeStruct((6 * _BQ,), jnp.float32),
        mesh=_mesh,
        scratch_types=[
            pltpu.VMEM((_QPW,), jnp.int32),
            pltpu.VMEM((_QPW,), jnp.float32),
        ],
    )
    return k(hx, hy, hz, vn[:_BV], vn[_BV:2 * _BV], vn[2 * _BV:], idx)


def _c2_body(g_ref, q_ref, o_ref):
    ax, ay, az = g_ref[0, 0], g_ref[0, 1], g_ref[0, 2]
    nx, ny, nz = g_ref[0, 3], g_ref[0, 4], g_ref[0, 5]
    dx = q_ref[0, 0] - ax
    dy = q_ref[0, 1] - ay
    dz = q_ref[0, 2] - az
    l2 = jnp.sqrt(dx * dx + dy * dy + dz * dz)
    dot = dx * nx + dy * ny + dz * nz
    coll = (dot < 0.0) & (l2 <= MAX_DIST)
    o_ref[pl.program_id(0), 0] = jnp.sum(coll.astype(jnp.float32)) / NG


def _c2_call(g, q_soa):
    return pl.pallas_call(
        _c2_body,
        grid=(B,),
        in_specs=[
            pl.BlockSpec((1, 6, NG), lambda b: (b, 0, 0)),
            pl.BlockSpec((1, 3, NG), lambda b: (b, 0, 0)),
        ],
        out_specs=pl.BlockSpec((B, 1), lambda b: (0, 0),
                               memory_space=pltpu.SMEM),
        out_shape=jax.ShapeDtypeStruct((B, 1), jnp.float32),
    )(g, q_soa)


def kernel(pred, h_state, faces, h_faces):
    del faces
    hx = h_state[:, :, 0].reshape(-1)
    hy = h_state[:, :, 1].reshape(-1)
    hz = h_state[:, :, 2].reshape(-1)
    fadj = h_faces + (jnp.arange(B, dtype=jnp.int32) * NH)[:, None, None]
    f0 = fadj[:, :, 0].reshape(-1)
    f1 = fadj[:, :, 1].reshape(-1)
    f2 = fadj[:, :, 2].reshape(-1)

    v9 = _a1_call(hx, hy, hz, f0, f1, f2)
    fn = _a2_call(v9)
    vn = _a3_call(fn, f0, f1, f2)

    qr = pred.astype(jnp.bfloat16).astype(jnp.float32)
    ar = h_state.astype(jnp.bfloat16).astype(jnp.float32)
    q = qr.transpose(0, 2, 1).reshape(B, 3, QS, QL)
    an2 = jnp.sum(h_state * h_state, axis=-1)
    a = jnp.concatenate([(-2.0 * ar).transpose(0, 2, 1), an2[:, None, :]],
                        axis=1)
    idx = _knn_call(q, a).reshape(-1)

    g = _c1_call(hx, hy, hz, vn, idx)
    q_soa = pred.transpose(0, 2, 1)
    return _c2_call(g.reshape(6, B, NG).transpose(1, 0, 2), q_soa)
